# baseline (speedup 1.0000x reference)
import functools

import jax
import jax.numpy as jnp
from jax.experimental import pallas as pl
from jax.experimental.pallas import tpu as pltpu

B = 1024
T_PAST = 48
T_FUT = 24
D_IN = 64
D_STATIC = 16
D_MODEL = 1024
D_FF = 2048
D_OUT = T_FUT * D_IN
E = 6
N_GT = 3
LAMBDA_LB = 0.01
BLK = 128
T_MAX = B // BLK + E - 1
PAD_ROWS = 2048


def _encoder_router_body(obs_ref, static_ref, we1_ref, be1_ref, ws_ref,
                         we2_ref, be2_ref, wr_ref, br_ref, nm_ref, ns_ref,
                         gt_ref, ctx_ref, masked_ref, gate_ref, sel_ref,
                         acc_ref):
    t = pl.program_id(0)

    @pl.when(t == 0)
    def _():
        acc_ref[...] = jnp.zeros_like(acc_ref)

    xt = obs_ref[:, t, :]
    xt = (xt - nm_ref[...]) / ns_ref[...]
    acc_ref[...] += jax.nn.gelu(
        jnp.dot(xt, we1_ref[...], preferred_element_type=jnp.float32)
        + be1_ref[...])

    @pl.when(t == T_PAST - 1)
    def _():
        pooled = acc_ref[...] / T_PAST + jnp.dot(
            static_ref[...], ws_ref[...], preferred_element_type=jnp.float32)
        ctx = jax.nn.gelu(
            jnp.dot(pooled, we2_ref[...], preferred_element_type=jnp.float32)
            + be2_ref[...])
        ctx_ref[...] = ctx
        logits = jnp.dot(ctx, wr_ref[...],
                         preferred_element_type=jnp.float32) + br_ref[...]
        probs = jax.nn.softmax(logits, axis=-1)
        gt = gt_ref[...]
        eids = jax.lax.broadcasted_iota(jnp.int32, (B, E), 1)
        allowed = (eids % N_GT) == gt
        masked = jnp.where(allowed, probs, 0.0)
        masked = masked / (jnp.sum(masked, axis=-1, keepdims=True) + 1e-9)
        masked_ref[...] = masked
        gmax = jnp.max(masked, axis=-1, keepdims=True)
        gate_ref[...] = jnp.broadcast_to(gmax, (B, 16))
        sel = jnp.min(jnp.where(masked >= gmax, eids, E), axis=-1,
                      keepdims=True)
        sel_ref[...] = sel


def _encoder_router(obs, static, we1, be1, ws, we2, be2, wr, br, nm, ns, gt):
    return pl.pallas_call(
        _encoder_router_body,
        grid=(T_PAST,),
        in_specs=[
            pl.BlockSpec((B, T_PAST, D_IN), lambda t: (0, 0, 0)),
            pl.BlockSpec((B, D_STATIC), lambda t: (0, 0)),
            pl.BlockSpec((D_IN, D_MODEL), lambda t: (0, 0)),
            pl.BlockSpec((1, D_MODEL), lambda t: (0, 0)),
            pl.BlockSpec((D_STATIC, D_MODEL), lambda t: (0, 0)),
            pl.BlockSpec((D_MODEL, D_MODEL), lambda t: (0, 0)),
            pl.BlockSpec((1, D_MODEL), lambda t: (0, 0)),
            pl.BlockSpec((D_MODEL, E), lambda t: (0, 0)),
            pl.BlockSpec((1, E), lambda t: (0, 0)),
            pl.BlockSpec((1, D_IN), lambda t: (0, 0)),
            pl.BlockSpec((1, D_IN), lambda t: (0, 0)),
            pl.BlockSpec((B, 1), lambda t: (0, 0)),
        ],
        out_specs=[
            pl.BlockSpec((B, D_MODEL), lambda t: (0, 0)),
            pl.BlockSpec((B, E), lambda t: (0, 0)),
            pl.BlockSpec((B, 16), lambda t: (0, 0)),
            pl.BlockSpec((B, 1), lambda t: (0, 0)),
        ],
        out_shape=[
            jax.ShapeDtypeStruct((B, D_MODEL), jnp.float32),
            jax.ShapeDtypeStruct((B, E), jnp.float32),
            jax.ShapeDtypeStruct((B, 16), jnp.float32),
            jax.ShapeDtypeStruct((B, 1), jnp.int32),
        ],
        scratch_shapes=[pltpu.VMEM((B, D_MODEL), jnp.float32)],
    )(obs, static, we1, be1, ws, we2, be2, wr, br, nm, ns, gt)


def _ffn_body(te_ref, ctx_ref, gate_ref, w1_ref, b1_ref, w2_ref, b2_ref,
              out_ref):
    x = ctx_ref[...]
    w1 = w1_ref[...].reshape(D_MODEL, D_FF)
    h = jax.nn.gelu(
        jnp.dot(x, w1, preferred_element_type=jnp.float32)
        + b1_ref[...].reshape(1, D_FF))
    w2 = w2_ref[...].reshape(D_FF, D_OUT)
    o = (jnp.dot(h, w2, preferred_element_type=jnp.float32)
         + b2_ref[...].reshape(1, D_OUT))
    out_ref[...] = o * gate_ref[:, :1]


def _ffn(tile_expert, ctx_pad, gate_pad, w1, b1, w2, b2):
    grid_spec = pltpu.PrefetchScalarGridSpec(
        num_scalar_prefetch=1,
        grid=(T_MAX,),
        in_specs=[
            pl.BlockSpec((BLK, D_MODEL), lambda t, te: (t, 0)),
            pl.BlockSpec((BLK, 16), lambda t, te: (t, 0)),
            pl.BlockSpec((1, D_MODEL, D_FF), lambda t, te: (te[t], 0, 0)),
            pl.BlockSpec((1, 1, D_FF), lambda t, te: (te[t], 0, 0)),
            pl.BlockSpec((1, D_FF, D_OUT), lambda t, te: (te[t], 0, 0)),
            pl.BlockSpec((1, 1, D_OUT), lambda t, te: (te[t], 0, 0)),
        ],
        out_specs=pl.BlockSpec((BLK, D_OUT), lambda t, te: (t, 0)),
    )
    return pl.pallas_call(
        _ffn_body,
        grid_spec=grid_spec,
        out_shape=jax.ShapeDtypeStruct((T_MAX * BLK, D_OUT), jnp.float32),
    )(tile_expert, ctx_pad, gate_pad, w1, b1, w2, b2)


def _loss_body(pred_ref, fut_ref, nm_ref, ns_ref, gt_ref, masked_ref,
               cnt_ref, loss_ref, sw_ref, sm_ref):
    i = pl.program_id(0)
    nb = pl.num_programs(0)

    @pl.when(i == 0)
    def _():
        sw_ref[...] = jnp.zeros_like(sw_ref)
        sm_ref[...] = jnp.zeros_like(sm_ref)

    fut_n = (fut_ref[...] - nm_ref[...]) / ns_ref[...]
    d = pred_ref[...] - fut_n
    mse = jnp.sum(d * d, axis=-1, keepdims=True) / D_OUT
    lw = jnp.where(gt_ref[...] == 0, 1.0, 5.0)
    sw_ref[...] += jnp.sum(mse * lw).reshape(1, 1)
    sm_ref[...] += jnp.sum(masked_ref[...], axis=0, keepdims=True)

    @pl.when(i == nb - 1)
    def _():
        frac = cnt_ref[0, :E].astype(jnp.float32) / B
        lb = LAMBDA_LB * E * jnp.sum(frac * sm_ref[0, :] / B)
        loss_ref[...] = (sw_ref[...] / B + lb)


def _loss(pred, fut, nm1536, ns1536, gt, masked, counts):
    blk = 128
    return pl.pallas_call(
        _loss_body,
        grid=(B // blk,),
        in_specs=[
            pl.BlockSpec((blk, D_OUT), lambda i: (i, 0)),
            pl.BlockSpec((blk, D_OUT), lambda i: (i, 0)),
            pl.BlockSpec((1, D_OUT), lambda i: (0, 0)),
            pl.BlockSpec((1, D_OUT), lambda i: (0, 0)),
            pl.BlockSpec((blk, 1), lambda i: (i, 0)),
            pl.BlockSpec((blk, E), lambda i: (i, 0)),
            pl.BlockSpec((1, 16), lambda i: (0, 0)),
        ],
        out_specs=pl.BlockSpec((1, 1), lambda i: (0, 0)),
        out_shape=jax.ShapeDtypeStruct((1, 1), jnp.float32),
        scratch_shapes=[pltpu.VMEM((1, 1), jnp.float32),
                        pltpu.VMEM((1, E), jnp.float32)],
    )(pred, fut, nm1536, ns1536, gt, masked, counts)


def kernel(observed_data, future_data, static_context, W_e1, b_e1, W_s, W_e2,
           b_e2, W_r, b_r, W1, b1, W2, b2, norm_mean, norm_std, phase_label):
    gt = phase_label.astype(jnp.int32).reshape(B, 1)
    ctx, masked, gate16, sel2 = _encoder_router(
        observed_data, static_context, W_e1, b_e1.reshape(1, D_MODEL), W_s,
        W_e2, b_e2.reshape(1, D_MODEL), W_r, b_r.reshape(1, E),
        norm_mean.reshape(1, D_IN), norm_std.reshape(1, D_IN), gt)

    sel = sel2[:, 0]
    onehot = (sel[:, None] == jnp.arange(E)[None, :])
    counts = jnp.sum(onehot, axis=0, dtype=jnp.int32)
    tiles = (counts + BLK - 1) // BLK
    tile_start = jnp.concatenate([jnp.zeros((1,), jnp.int32),
                                  jnp.cumsum(tiles)[:-1]])
    rids = jnp.arange(B, dtype=jnp.int32)
    rank = jnp.sum((sel[None, :] == sel[:, None])
                   & (rids[None, :] < rids[:, None]), axis=1,
                   dtype=jnp.int32)
    slot = tile_start[sel] * BLK + rank
    src = jnp.zeros((PAD_ROWS,), jnp.int32).at[slot].set(rids)
    t_ids = jnp.arange(16, dtype=jnp.int32)
    tile_cum = jnp.cumsum(tiles)
    tile_expert = jnp.minimum(
        jnp.sum(t_ids[:, None] >= tile_cum[None, :], axis=1,
                dtype=jnp.int32), E - 1)

    ctx_pad = jnp.take(ctx, src, axis=0)
    gate_pad = jnp.take(gate16, src, axis=0)

    pred_pad = _ffn(tile_expert, ctx_pad, gate_pad, W1,
                    b1.reshape(E, 1, D_FF), W2, b2.reshape(E, 1, D_OUT))
    pred_rows = jnp.take(pred_pad, slot, axis=0)

    nm1536 = jnp.tile(norm_mean, T_FUT).reshape(1, D_OUT)
    ns1536 = jnp.tile(norm_std, T_FUT).reshape(1, D_OUT)
    counts16 = jnp.zeros((1, 16), jnp.int32).at[0, :E].set(counts)
    loss = _loss(pred_rows, future_data.reshape(B, D_OUT), nm1536, ns1536,
                 gt, masked, counts16)

    pred = pred_rows.reshape(B, T_FUT, D_IN)
    return loss[0, 0], pred, masked
